# baseline (speedup 1.0000x reference)
import functools

import jax
import jax.numpy as jnp
from jax import lax
from jax.experimental import pallas as pl
from jax.experimental.pallas import tpu as pltpu
from jax.experimental.pallas import tpu_sc as plsc

N_NODES = 10000
N_EDGES = 320000
D = 128

CHUNK = 128
NCHUNKS = N_EDGES // CHUNK
NC, NS = 2, 16
NW = NC * NS
FULL_PER_TILE = 80
W15_CNT = 48
LAST_CNT = 52
TAIL_ROWS = 56
ACC_ROWS = 10112
ROWS_PER_TILE = ACC_ROWS // NS
NBUF = 2


def _sc_body(msgs_hbm, idx_hbm, tail_hbm, out_hbm,
             idx_v, msg_v, acc_s, ld_sem, sc_sem):
    c = lax.axis_index("c")
    s = lax.axis_index("s")
    w = c * NS + s

    cnt = jnp.where(w == NS - 1, W15_CNT,
                    jnp.where(w == NW - 1, LAST_CNT, FULL_PER_TILE))
    start = jnp.where(w < NS, w * FULL_PER_TILE,
                      w * FULL_PER_TILE - (FULL_PER_TILE - W15_CNT))

    @pl.when(jnp.logical_and(w != NS - 1, w != NW - 1))
    def _load_idx_full():
        pltpu.async_copy(idx_hbm.at[pl.ds(start, FULL_PER_TILE)], idx_v,
                         sc_sem.at[0])

    @pl.when(w == NS - 1)
    def _load_idx_w15():
        pltpu.async_copy(idx_hbm.at[pl.ds(start, W15_CNT)],
                         idx_v.at[pl.ds(0, W15_CNT)], sc_sem.at[0])

    @pl.when(w == NW - 1)
    def _load_idx_last():
        pltpu.async_copy(tail_hbm, idx_v.at[pl.ds(0, TAIL_ROWS)],
                         sc_sem.at[0])

    zero16 = jnp.zeros((16,), jnp.float32)

    def zrow(i, carry):
        for j in range(D // 16):
            msg_v[0, i, pl.ds(16 * j, 16)] = zero16
        return carry

    lax.fori_loop(0, CHUNK, zrow, 0)

    row0 = s * ROWS_PER_TILE
    nfull = ROWS_PER_TILE // CHUNK
    rem = ROWS_PER_TILE - nfull * CHUNK
    for b in range(nfull):
        pltpu.async_copy(msg_v.at[0], acc_s.at[pl.ds(row0 + b * CHUNK, CHUNK)],
                         sc_sem.at[1])
    pltpu.async_copy(msg_v.at[0].at[pl.ds(0, rem)],
                     acc_s.at[pl.ds(row0 + nfull * CHUNK, rem)],
                     sc_sem.at[1])
    pltpu.async_copy(msgs_hbm.at[start + 1], msg_v.at[1], ld_sem.at[1])

    for b in range(nfull):
        pltpu.make_async_copy(msg_v.at[0],
                              acc_s.at[pl.ds(row0 + b * CHUNK, CHUNK)],
                              sc_sem.at[1]).wait()
    pltpu.make_async_copy(msg_v.at[0].at[pl.ds(0, rem)],
                          acc_s.at[pl.ds(row0 + nfull * CHUNK, rem)],
                          sc_sem.at[1]).wait()
    pltpu.async_copy(msgs_hbm.at[start], msg_v.at[0], ld_sem.at[0])

    @pl.when(jnp.logical_and(w != NS - 1, w != NW - 1))
    def _wait_idx_full():
        pltpu.make_async_copy(idx_hbm.at[pl.ds(start, FULL_PER_TILE)], idx_v,
                              sc_sem.at[0]).wait()

    @pl.when(w == NS - 1)
    def _wait_idx_w15():
        pltpu.make_async_copy(idx_hbm.at[pl.ds(start, W15_CNT)],
                              idx_v.at[pl.ds(0, W15_CNT)], sc_sem.at[0]).wait()

    @pl.when(w == NW - 1)
    def _wait_idx_last():
        pltpu.make_async_copy(tail_hbm, idx_v.at[pl.ds(0, TAIL_ROWS)],
                              sc_sem.at[0]).wait()

    plsc.subcore_barrier()

    rounds = cnt // NBUF

    def round_body(g, carry):
        for b in range(NBUF):
            i = g * NBUF + b
            pltpu.make_async_copy(msgs_hbm.at[start + i], msg_v.at[b],
                                  ld_sem.at[b]).wait()
            desc = pltpu.async_copy(msg_v.at[b], acc_s.at[idx_v.at[i]],
                                    sc_sem.at[b], add=True)

            @pl.when(g < rounds - 1)
            def _reload():
                desc.wait()
                pltpu.async_copy(msgs_hbm.at[start + i + NBUF], msg_v.at[b],
                                 ld_sem.at[b])
        return carry

    lax.fori_loop(0, rounds, round_body, 0)

    for b in range(NBUF):
        i = cnt - NBUF + b
        pltpu.make_async_copy(msg_v.at[b], acc_s.at[idx_v.at[i]],
                              sc_sem.at[b]).wait()
    plsc.subcore_barrier()

    pltpu.sync_copy(acc_s.at[pl.ds(row0, ROWS_PER_TILE)],
                    out_hbm.at[c, pl.ds(row0, ROWS_PER_TILE)])


_scatter_sc = functools.partial(
    pl.kernel,
    mesh=plsc.VectorSubcoreMesh(core_axis_name="c", subcore_axis_name="s"),
    out_type=jax.ShapeDtypeStruct((NC, ACC_ROWS, D), jnp.float32),
    scratch_types=[
        pltpu.VMEM((FULL_PER_TILE, CHUNK), jnp.int32),
        pltpu.VMEM((NBUF, CHUNK, D), jnp.float32),
        pltpu.VMEM_SHARED((ACC_ROWS, D), jnp.float32),
        pltpu.SemaphoreType.DMA((NBUF,)),
        pltpu.SemaphoreType.DMA((NBUF,)),
    ],
)(_sc_body)


def _combine_body(p_ref, o_ref):
    o_ref[...] = p_ref[0] + p_ref[1]


def _combine(partials):
    return pl.pallas_call(
        _combine_body,
        out_shape=jax.ShapeDtypeStruct((N_NODES, D), jnp.float32),
        grid=(5,),
        in_specs=[pl.BlockSpec((NC, N_NODES // 5, D), lambda i: (0, i, 0))],
        out_specs=pl.BlockSpec((N_NODES // 5, D), lambda i: (i, 0)),
    )(partials)


@jax.jit
def kernel(messages, edge_index):
    dst = edge_index[1].astype(jnp.int32).reshape(NCHUNKS, CHUNK)
    tail = jnp.pad(dst[NCHUNKS - LAST_CNT:], ((0, TAIL_ROWS - LAST_CNT), (0, 0)))
    msgs = messages.reshape(NCHUNKS, CHUNK, D)
    partials = _scatter_sc(msgs, dst, tail)
    return _combine(partials)
